# baseline (speedup 1.0000x reference)
import functools

import jax
import jax.numpy as jnp
from jax.experimental import pallas as pl
from jax.experimental.pallas import tpu as pltpu


def _round_up(x, m):
    return (x + m - 1) // m * m


def _fused_kernel(g_hbm, h_hbm, wt_ref, b_ref, o_ref,
                  g_buf, h_buf, g_sem, h_sem, *, rt, n_tiles):
    i = pl.program_id(0)
    row0 = i * (rt * n_tiles)

    def start_tile(t, slot):
        pltpu.make_async_copy(
            g_hbm.at[pl.ds(row0 + t * rt, rt)], g_buf.at[slot], g_sem.at[slot]
        ).start()

    def wait_tile(slot):
        pltpu.make_async_copy(
            g_hbm.at[pl.ds(0, rt)], g_buf.at[slot], g_sem.at[slot]
        ).wait()

    pltpu.make_async_copy(h_hbm, h_buf, h_sem).start()
    start_tile(0, 0)

    for t in range(n_tiles):
        if t + 1 < n_tiles:
            start_tile(t + 1, (t + 1) % 2)
        wait_tile(t % 2)
        if t == 0:
            pltpu.make_async_copy(h_hbm, h_buf, h_sem).wait()
        acc = jnp.dot(g_buf[t % 2], h_buf[...],
                      preferred_element_type=jnp.float32)
        o_ref[pl.ds(t * rt, rt), :] = jnp.maximum(
            jnp.dot(acc, wt_ref[...], preferred_element_type=jnp.float32)
            + b_ref[...],
            0.0,
        ).astype(o_ref.dtype)


def kernel(g, h, w, b):
    n = g.shape[0]
    out_dim, in_dim = w.shape
    assert g.shape == (n, n) and h.shape == (n, in_dim)

    rt = 512 if n % 1024 == 0 else 256
    n_pad = _round_up(n, 2 * rt)
    n_tiles = n_pad // (2 * rt)
    in_pad = _round_up(in_dim, 128)
    out_pad = _round_up(out_dim, 128)

    dtype = h.dtype
    g_p = jnp.pad(g, ((0, n_pad - n), (0, n_pad - n))).astype(dtype)
    h_p = jnp.pad(h, ((0, n_pad - n), (0, in_pad - in_dim))).astype(dtype)
    wt_p = jnp.pad(w.T, ((0, in_pad - in_dim), (0, out_pad - out_dim))).astype(dtype)
    b_p = jnp.pad(b.reshape(1, -1), ((0, 0), (0, out_pad - out_dim))).astype(jnp.float32)

    cost = pl.CostEstimate(
        flops=2 * n_pad * n_pad * in_pad + 2 * n_pad * in_pad * out_pad,
        transcendentals=0,
        bytes_accessed=4 * (n_pad * n_pad + n_pad * in_pad
                            + in_pad * out_pad + n_pad * out_pad),
    )
    body = functools.partial(_fused_kernel, rt=rt, n_tiles=n_tiles)
    out_p = pl.pallas_call(
        body,
        out_shape=jax.ShapeDtypeStruct((n_pad, out_pad), dtype),
        grid=(2,),
        in_specs=[
            pl.BlockSpec(memory_space=pl.ANY),
            pl.BlockSpec(memory_space=pl.ANY),
            pl.BlockSpec((in_pad, out_pad), lambda i: (0, 0)),
            pl.BlockSpec((1, out_pad), lambda i: (0, 0)),
        ],
        out_specs=pl.BlockSpec((n_pad // 2, out_pad), lambda i: (i, 0)),
        scratch_shapes=[
            pltpu.VMEM((2, rt, n_pad), dtype),
            pltpu.VMEM((n_pad, in_pad), dtype),
            pltpu.SemaphoreType.DMA((2,)),
            pltpu.SemaphoreType.DMA,
        ],
        compiler_params=pltpu.CompilerParams(
            dimension_semantics=("parallel",),
            vmem_limit_bytes=56 * 1024 * 1024,
        ),
        cost_estimate=cost,
    )(g_p, h_p, wt_p, b_p)

    return out_p[:n, :out_dim]


# speedup vs baseline: 1.1165x; 1.1165x over previous
import jax
import jax.numpy as jnp
from jax.experimental import pallas as pl
from jax.experimental.pallas import tpu as pltpu


def _round_up(x, m):
    return (x + m - 1) // m * m


def _fused_kernel(ga_ref, gb_ref, h_ref, wt_ref, b_ref, o_ref):
    half = ga_ref.shape[0]
    for g_ref, sl in ((ga_ref, pl.ds(0, half)), (gb_ref, pl.ds(half, half))):
        acc = jnp.dot(g_ref[...], h_ref[...], preferred_element_type=jnp.float32)
        o_ref[sl, :] = jnp.maximum(
            jnp.dot(acc, wt_ref[...], preferred_element_type=jnp.float32)
            + b_ref[...],
            0.0,
        ).astype(o_ref.dtype)


def kernel(g, h, w, b):
    n = g.shape[0]
    out_dim, in_dim = w.shape
    assert g.shape == (n, n) and h.shape == (n, in_dim)

    tm = 512 if n % 512 == 0 else 256
    n_pad = _round_up(n, tm)
    in_pad = _round_up(in_dim, 128)
    out_pad = _round_up(out_dim, 128)
    half = tm // 2

    dtype = h.dtype
    g_p = jnp.pad(g, ((0, n_pad - n), (0, n_pad - n))).astype(dtype)
    h_p = jnp.pad(h, ((0, n_pad - n), (0, in_pad - in_dim))).astype(dtype)
    wt_p = jnp.pad(w.T, ((0, in_pad - in_dim), (0, out_pad - out_dim))).astype(dtype)
    b_p = jnp.pad(b.reshape(1, -1), ((0, 0), (0, out_pad - out_dim))).astype(jnp.float32)

    cost = pl.CostEstimate(
        flops=2 * n_pad * n_pad * in_pad + 2 * n_pad * in_pad * out_pad,
        transcendentals=0,
        bytes_accessed=4 * (n_pad * n_pad + n_pad * in_pad
                            + in_pad * out_pad + n_pad * out_pad),
    )
    out_p = pl.pallas_call(
        _fused_kernel,
        out_shape=jax.ShapeDtypeStruct((n_pad, out_pad), dtype),
        grid=(n_pad // tm,),
        in_specs=[
            pl.BlockSpec((half, n_pad), lambda i: (2 * i, 0)),
            pl.BlockSpec((half, n_pad), lambda i: (2 * i + 1, 0)),
            pl.BlockSpec((n_pad, in_pad), lambda i: (0, 0)),
            pl.BlockSpec((in_pad, out_pad), lambda i: (0, 0)),
            pl.BlockSpec((1, out_pad), lambda i: (0, 0)),
        ],
        out_specs=pl.BlockSpec((tm, out_pad), lambda i: (i, 0)),
        compiler_params=pltpu.CompilerParams(
            dimension_semantics=("parallel",),
            vmem_limit_bytes=56 * 1024 * 1024,
        ),
        cost_estimate=cost,
    )(g_p, g_p, h_p, wt_p, b_p)

    return out_p[:n, :out_dim]


# speedup vs baseline: 1.2767x; 1.1435x over previous
import jax
import jax.numpy as jnp
from jax.experimental import pallas as pl
from jax.experimental.pallas import tpu as pltpu


def _round_up(x, m):
    return (x + m - 1) // m * m


def _fused_kernel(g_ref, h_ref, wt_ref, b_ref, o_ref):
    t = jnp.dot(g_ref[...], h_ref[...], preferred_element_type=jnp.float32)
    o_ref[...] = jnp.maximum(
        jnp.dot(t, wt_ref[...], preferred_element_type=jnp.float32)
        + b_ref[...],
        0.0,
    ).astype(o_ref.dtype)


def kernel(g, h, w, b):
    n = g.shape[0]
    out_dim, in_dim = w.shape
    assert g.shape == (n, n) and h.shape == (n, in_dim)

    tm = 512 if n % 512 == 0 else 256
    n_pad = _round_up(n, tm)
    in_pad = _round_up(in_dim, 128)
    out_pad = _round_up(out_dim, 128)

    dtype = h.dtype
    g_p = jnp.pad(g, ((0, n_pad - n), (0, n_pad - n))).astype(dtype)
    h_p = jnp.pad(h, ((0, n_pad - n), (0, in_pad - in_dim))).astype(dtype)
    wt_p = jnp.pad(w.T, ((0, in_pad - in_dim), (0, out_pad - out_dim))).astype(dtype)
    b_p = jnp.pad(b.reshape(1, -1), ((0, 0), (0, out_pad - out_dim))).astype(jnp.float32)

    cost = pl.CostEstimate(
        flops=2 * n_pad * n_pad * in_pad + 2 * n_pad * in_pad * out_pad,
        transcendentals=0,
        bytes_accessed=4 * (n_pad * n_pad + n_pad * in_pad
                            + in_pad * out_pad + n_pad * out_pad),
    )
    out_p = pl.pallas_call(
        _fused_kernel,
        out_shape=jax.ShapeDtypeStruct((n_pad, out_pad), dtype),
        grid=(n_pad // tm,),
        in_specs=[
            pl.BlockSpec((tm, n_pad), lambda i: (i, 0)),
            pl.BlockSpec((n_pad, in_pad), lambda i: (0, 0)),
            pl.BlockSpec((in_pad, out_pad), lambda i: (0, 0)),
            pl.BlockSpec((1, out_pad), lambda i: (0, 0)),
        ],
        out_specs=pl.BlockSpec((tm, out_pad), lambda i: (i, 0)),
        compiler_params=pltpu.CompilerParams(
            dimension_semantics=("parallel",),
            vmem_limit_bytes=56 * 1024 * 1024,
        ),
        cost_estimate=cost,
    )(g_p, h_p, wt_p, b_p)

    return out_p[:n, :out_dim]
